# baseline (speedup 1.0000x reference)
import jax
import jax.numpy as jnp
from jax import lax
from jax.experimental import pallas as pl

_K = 512
_N = 16384
_ROWS = 32
_LO0 = -2139095041
_HI0 = 2139095041


def _unsort(m):
    return jnp.where(m >= 0, m, m ^ jnp.int32(0x7FFFFFFF))


def _topk_mask_body(x_ref, o_ref):
    x = x_ref[...]
    r = x.shape[0]
    lo0 = jnp.full((r, 1), _LO0, jnp.int32)
    hi0 = jnp.full((r, 1), _HI0, jnp.int32)
    ones = jnp.ones((_N, 1), jnp.float32)

    def body(_, carry):
        lo, hi = carry
        mid = (lo & hi) + ((lo ^ hi) >> 1)
        midf = lax.bitcast_convert_type(_unsort(mid), jnp.float32)
        maskf = (x >= midf).astype(jnp.float32)
        cnt = jnp.dot(maskf, ones,
                      precision=jax.lax.Precision.HIGHEST)
        ge = cnt >= jnp.float32(_K)
        return jnp.where(ge, mid, lo), jnp.where(ge, hi, mid)

    lo, _ = lax.fori_loop(0, 32, body, (lo0, hi0))
    tf = lax.bitcast_convert_type(_unsort(lo), jnp.float32)
    o_ref[...] = (x >= tf).astype(jnp.float32)


def kernel(scores):
    bsz, nmax, _, ens = scores.shape
    s = jnp.transpose(scores, (0, 3, 1, 2)).reshape(bsz * ens, nmax * nmax)
    out = pl.pallas_call(
        _topk_mask_body,
        grid=(s.shape[0] // _ROWS,),
        in_specs=[pl.BlockSpec((_ROWS, _N), lambda r: (r, 0))],
        out_specs=pl.BlockSpec((_ROWS, _N), lambda r: (r, 0)),
        out_shape=jax.ShapeDtypeStruct(s.shape, jnp.float32),
    )(s)
    out = out.reshape(bsz, ens, nmax, nmax)
    return jnp.transpose(out, (0, 2, 3, 1))


# speedup vs baseline: 8.6106x; 8.6106x over previous
import jax
import jax.numpy as jnp
from jax import lax
from jax.experimental import pallas as pl

_K = 512
_N = 16384
_ROWS = 64
_LO0 = -2139095041
_HI0 = 2139095041


def _unsort(m):
    return jnp.where(m >= 0, m, m ^ jnp.int32(0x7FFFFFFF))


def _topk_mask_body(x_ref, o_ref):
    x = x_ref[...]
    r = x.shape[0]
    lo0 = jnp.full((r, 1), _LO0, jnp.int32)
    hi0 = jnp.full((r, 1), _HI0, jnp.int32)
    def body(_, carry):
        lo, hi = carry
        mid = (lo & hi) + ((lo ^ hi) >> 1)
        midf = lax.bitcast_convert_type(_unsort(mid), jnp.float32)
        cnt = jnp.sum((x >= midf).astype(jnp.int32), axis=1, keepdims=True)
        ge = cnt >= _K
        return jnp.where(ge, mid, lo), jnp.where(ge, hi, mid)

    lo, _ = lax.fori_loop(0, 32, body, (lo0, hi0))
    tf = lax.bitcast_convert_type(_unsort(lo), jnp.float32)
    o_ref[...] = (x >= tf).astype(jnp.float32)


def kernel(scores):
    bsz, nmax, _, ens = scores.shape
    s = jnp.transpose(scores, (0, 3, 1, 2)).reshape(bsz * ens, nmax * nmax)
    out = pl.pallas_call(
        _topk_mask_body,
        grid=(s.shape[0] // _ROWS,),
        in_specs=[pl.BlockSpec((_ROWS, _N), lambda r: (r, 0))],
        out_specs=pl.BlockSpec((_ROWS, _N), lambda r: (r, 0)),
        out_shape=jax.ShapeDtypeStruct(s.shape, jnp.float32),
    )(s)
    out = out.reshape(bsz, ens, nmax, nmax)
    return jnp.transpose(out, (0, 2, 3, 1))


# speedup vs baseline: 9.6347x; 1.1189x over previous
import jax
import jax.numpy as jnp
from jax import lax
from jax.experimental import pallas as pl

_K = 512
_N = 16384
_ROWS = 128
_LO0 = -2139095041
_HI0 = 2139095041


def _unsort(m):
    return jnp.where(m >= 0, m, m ^ jnp.int32(0x7FFFFFFF))


def _topk_mask_body(x_ref, o_ref):
    x = x_ref[...]
    r = x.shape[0]
    lo0 = jnp.full((r, 1), _LO0, jnp.int32)
    hi0 = jnp.full((r, 1), _HI0, jnp.int32)
    def body(_, carry):
        lo, hi = carry
        mid = (lo & hi) + ((lo ^ hi) >> 1)
        midf = lax.bitcast_convert_type(_unsort(mid), jnp.float32)
        cnt = jnp.sum((x >= midf).astype(jnp.int32), axis=1, keepdims=True)
        ge = cnt >= _K
        return jnp.where(ge, mid, lo), jnp.where(ge, hi, mid)

    lo, _ = lax.fori_loop(0, 32, body, (lo0, hi0))
    tf = lax.bitcast_convert_type(_unsort(lo), jnp.float32)
    o_ref[...] = (x >= tf).astype(jnp.float32)


def kernel(scores):
    bsz, nmax, _, ens = scores.shape
    s = jnp.transpose(scores, (0, 3, 1, 2)).reshape(bsz * ens, nmax * nmax)
    out = pl.pallas_call(
        _topk_mask_body,
        grid=(s.shape[0] // _ROWS,),
        in_specs=[pl.BlockSpec((_ROWS, _N), lambda r: (r, 0))],
        out_specs=pl.BlockSpec((_ROWS, _N), lambda r: (r, 0)),
        out_shape=jax.ShapeDtypeStruct(s.shape, jnp.float32),
    )(s)
    out = out.reshape(bsz, ens, nmax, nmax)
    return jnp.transpose(out, (0, 2, 3, 1))
